# baseline (speedup 1.0000x reference)
import functools

import jax
import jax.numpy as jnp
from jax import lax
from jax.experimental import pallas as pl
from jax.experimental.pallas import tpu as pltpu
from jax.experimental.pallas import tpu_sc as plsc

N = 268
E = 8576
L = 16
NP = 272


def _sc_build_adjacency(ei, ea):
    g = ea.shape[0]
    info = plsc.get_sparse_core_info()
    nc, ns = info.num_cores, info.num_subcores
    assert nc * ns == g

    mesh = plsc.VectorSubcoreMesh(core_axis_name="c", subcore_axis_name="s")

    @functools.partial(
        pl.kernel,
        mesh=mesh,
        out_type=jax.ShapeDtypeStruct((g, N, NP), jnp.float32),
        scratch_types=[
            pltpu.VMEM((E,), jnp.int32),
            pltpu.VMEM((E,), jnp.int32),
            pltpu.VMEM((E,), jnp.float32),
            pltpu.VMEM((N, NP), jnp.float32),
            pltpu.SemaphoreType.DMA,
        ],
        compiler_params=pltpu.CompilerParams(needs_layout_passes=False),
    )
    def scatter_kernel(ei_h, ea_h, o_h, src_v, dst_v, ew_v, a_v, sem):
        w = lax.axis_index("s") * nc + lax.axis_index("c")
        zeros16 = jnp.zeros((L,), jnp.float32)

        cps = [pltpu.async_copy(ei_h.at[2 * w], src_v, sem),
               pltpu.async_copy(ei_h.at[2 * w + 1], dst_v, sem),
               pltpu.async_copy(ea_h.at[w], ew_v, sem)]

        @plsc.parallel_loop(0, N)
        def _zero(i):
            for j in range(NP // L):
                a_v[i, pl.ds(j * L, L)] = zeros16

        for cp in cps:
            cp.wait()

        @plsc.parallel_loop(0, E // L, unroll=4)
        def _scat(i):
            s = src_v[pl.ds(i * L, L)]
            d = dst_v[pl.ds(i * L, L)]
            v = ew_v[pl.ds(i * L, L)]
            plsc.addupdate_scatter(a_v, [d, s], v)

        pltpu.sync_copy(a_v, o_h.at[w])

    return scatter_kernel(ei, ea)


def _tc_body(x_ref, a_ref, w1_ref, b1_ref, w4_ref, b4_ref, wc1_ref, bc1_ref,
             wc2_ref, bc2_ref, out_ref):
    f32 = jnp.float32
    hi = lax.Precision.DEFAULT

    def mm(p, q):
        return lax.dot_general(p, q, (((1,), (0,)), ((), ())),
                               precision=hi, preferred_element_type=f32)

    gb = x_ref.shape[0]
    ks = range(gb)
    xs = [x_ref[k] for k in ks]
    avs = [a_ref[k][:, :N] for k in ks]

    s_mats = []
    for k in ks:
        deg_r = jnp.sum(avs[k], axis=0, keepdims=True)
        dis_r = jnp.where(deg_r > 0,
                          lax.rsqrt(jnp.where(deg_r > 0, deg_r, 1.0)), 0.0)
        dis_c = jnp.transpose(dis_r)
        s_mats.append((dis_c * avs[k]) * dis_r)

    p = [mm(xs[k], w1_ref[...]) for k in ks]
    sp2 = [mm(s_mats[k], p[k][:, 64:96]) for k in ks]
    sall = [mm(s_mats[k], p[k][:, 32:64] - 2.0 * sp2[k]) for k in ks]
    h = [jnp.maximum(p[k][:, 0:32] - sall[k] + b1_ref[...], 0.0) for k in ks]

    q = [mm(h[k], w4_ref[...]) for k in ks]
    sq2 = [mm(s_mats[k], q[k][:, 64:96]) for k in ks]
    sall2 = [mm(s_mats[k], q[k][:, 32:64] - 2.0 * sq2[k]) for k in ks]
    z = [q[k][:, 0:32] - sall2[k] + b4_ref[...] for k in ks]

    zc = [lax.dot_general(z[k], wc1_ref[...], (((0,), (0,)), ((), ())),
                          precision=hi, preferred_element_type=f32)
          for k in ks]
    zc = [jnp.maximum(zc[k] + bc1_ref[...], 0.0) for k in ks]
    for k in ks:
        out_ref[k] = mm(zc[k], wc2_ref[...]) + bc2_ref[...]


def _tc_dense(x_all, a_all, w1c, b1, w4c, b4, wc1, bc1, wc2, bc2,
              interpret=False):
    g = x_all.shape[0]
    nclass = 32
    nout = wc2.shape[-1]

    gb = 4
    full = lambda shape: pl.BlockSpec(shape, lambda i: (0,) * len(shape))
    grid_spec = pl.GridSpec(
        grid=(g // gb,),
        in_specs=[
            pl.BlockSpec((gb, N, N), lambda i: (i, 0, 0)),
            pl.BlockSpec((gb, N, NP), lambda i: (i, 0, 0)),
            full(w1c.shape),
            full((1, b1.shape[0])),
            full(w4c.shape),
            full((1, b4.shape[0])),
            full(wc1.shape),
            full((1, bc1.shape[0])),
            full(wc2.shape),
            full((1, bc2.shape[0])),
        ],
        out_specs=pl.BlockSpec((gb, nclass, nout), lambda i: (i, 0, 0)),
    )
    return pl.pallas_call(
        _tc_body,
        grid_spec=grid_spec,
        out_shape=jax.ShapeDtypeStruct((g, nclass, nout), jnp.float32),
        interpret=interpret,
    )(x_all, a_all, w1c, b1.reshape(1, -1), w4c, b4.reshape(1, -1),
      wc1, bc1.reshape(1, -1), wc2, bc2.reshape(1, -1))


def kernel(x1, edge_index1, edge_attr1, x2, edge_index2, edge_attr2,
           W1, b1, W4, b4, Wc1, bc1, Wc2, bc2):
    g = x1.shape[0]
    ei1 = edge_index1.astype(jnp.int32).reshape(2 * g, E)
    ei2 = edge_index2.astype(jnp.int32).reshape(2 * g, E)
    ea1 = edge_attr1.astype(jnp.float32)
    ea2 = edge_attr2.astype(jnp.float32)

    a1 = _sc_build_adjacency(ei1, ea1)
    a2 = _sc_build_adjacency(ei2, ea2)

    w1c = jnp.concatenate([W1[0] - W1[2], W1[1], W1[2]], axis=1)
    w4c = jnp.concatenate([W4[0] - W4[2], W4[1], W4[2]], axis=1)

    out1 = _tc_dense(x1, a1, w1c, b1, w4c, b4, Wc1, bc1, Wc2, bc2)
    out2 = _tc_dense(x2, a2, w1c, b1, w4c, b4, Wc1, bc1, Wc2, bc2)
    return out1, out2


# speedup vs baseline: 1.0951x; 1.0951x over previous
import functools

import jax
import jax.numpy as jnp
from jax import lax
from jax.experimental import pallas as pl
from jax.experimental.pallas import tpu as pltpu
from jax.experimental.pallas import tpu_sc as plsc

N = 268
E = 8576
L = 16
NP = 272


def _sc_build_adjacency(ei, ea):
    g = ea.shape[0]
    info = plsc.get_sparse_core_info()
    nc, ns = info.num_cores, info.num_subcores
    assert nc * ns == g

    mesh = plsc.VectorSubcoreMesh(core_axis_name="c", subcore_axis_name="s")

    @functools.partial(
        pl.kernel,
        mesh=mesh,
        out_type=jax.ShapeDtypeStruct((g * NP, NP), jnp.float32),
        scratch_types=[
            pltpu.VMEM((E,), jnp.int32),
            pltpu.VMEM((E,), jnp.int32),
            pltpu.VMEM((E,), jnp.float32),
            pltpu.VMEM((NP, NP), jnp.float32),
            pltpu.SemaphoreType.DMA,
        ],
        compiler_params=pltpu.CompilerParams(needs_layout_passes=False),
    )
    def scatter_kernel(ei_h, ea_h, o_h, src_v, dst_v, ew_v, a_v, sem):
        w = lax.axis_index("s") * nc + lax.axis_index("c")
        zeros16 = jnp.zeros((L,), jnp.float32)

        cps = [pltpu.async_copy(ei_h.at[w, 0], src_v, sem),
               pltpu.async_copy(ei_h.at[w, 1], dst_v, sem),
               pltpu.async_copy(ea_h.at[w], ew_v, sem)]

        @plsc.parallel_loop(0, NP)
        def _zero(i):
            for j in range(NP // L):
                a_v[i, pl.ds(j * L, L)] = zeros16

        for cp in cps:
            cp.wait()

        @plsc.parallel_loop(0, E // L, unroll=4)
        def _scat(i):
            s = src_v[pl.ds(i * L, L)]
            d = dst_v[pl.ds(i * L, L)]
            v = ew_v[pl.ds(i * L, L)]
            plsc.addupdate_scatter(a_v, [d, s], v)

        pltpu.sync_copy(a_v, o_h.at[pl.ds(w * NP, NP)])

    return scatter_kernel(ei, ea)


def _tc_body(x_ref, a_ref, w1_ref, b1_ref, w4_ref, b4_ref, wc1_ref, bc1_ref,
             wc2_ref, bc2_ref, out_ref):
    f32 = jnp.float32
    hi = lax.Precision.DEFAULT

    def mm(p, q):
        return lax.dot_general(p, q, (((1,), (0,)), ((), ())),
                               precision=hi, preferred_element_type=f32)

    gb = x_ref.shape[0]
    ks = range(gb)
    xs = [x_ref[k] for k in ks]
    avs = [a_ref[k * NP:k * NP + N, :N] for k in ks]

    s_mats = []
    for k in ks:
        deg_r = jnp.sum(avs[k], axis=0, keepdims=True)
        dis_r = jnp.where(deg_r > 0,
                          lax.rsqrt(jnp.where(deg_r > 0, deg_r, 1.0)), 0.0)
        dis_c = jnp.transpose(dis_r)
        s_mats.append((dis_c * avs[k]) * dis_r)

    p = [mm(xs[k], w1_ref[...]) for k in ks]
    sp2 = [mm(s_mats[k], p[k][:, 64:96]) for k in ks]
    sall = [mm(s_mats[k], p[k][:, 32:64] - 2.0 * sp2[k]) for k in ks]
    h = [jnp.maximum(p[k][:, 0:32] - sall[k] + b1_ref[...], 0.0) for k in ks]

    q = [mm(h[k], w4_ref[...]) for k in ks]
    sq2 = [mm(s_mats[k], q[k][:, 64:96]) for k in ks]
    sall2 = [mm(s_mats[k], q[k][:, 32:64] - 2.0 * sq2[k]) for k in ks]
    z = [q[k][:, 0:32] - sall2[k] + b4_ref[...] for k in ks]

    zc = [lax.dot_general(z[k], wc1_ref[...], (((0,), (0,)), ((), ())),
                          precision=hi, preferred_element_type=f32)
          for k in ks]
    zc = [jnp.maximum(zc[k] + bc1_ref[...], 0.0) for k in ks]
    for k in ks:
        out_ref[k] = mm(zc[k], wc2_ref[...]) + bc2_ref[...]


def _tc_dense(x_all, a_all, w1c, b1, w4c, b4, wc1, bc1, wc2, bc2,
              interpret=False):
    g = x_all.shape[0]
    nclass = 32
    nout = wc2.shape[-1]

    gb = 4
    full = lambda shape: pl.BlockSpec(shape, lambda i: (0,) * len(shape))
    grid_spec = pl.GridSpec(
        grid=(g // gb,),
        in_specs=[
            pl.BlockSpec((gb, N, N), lambda i: (i, 0, 0)),
            pl.BlockSpec((gb * NP, NP), lambda i: (i, 0)),
            full(w1c.shape),
            full((1, b1.shape[0])),
            full(w4c.shape),
            full((1, b4.shape[0])),
            full(wc1.shape),
            full((1, bc1.shape[0])),
            full(wc2.shape),
            full((1, bc2.shape[0])),
        ],
        out_specs=pl.BlockSpec((gb, nclass, nout), lambda i: (i, 0, 0)),
    )
    return pl.pallas_call(
        _tc_body,
        grid_spec=grid_spec,
        out_shape=jax.ShapeDtypeStruct((g, nclass, nout), jnp.float32),
        interpret=interpret,
    )(x_all, a_all, w1c, b1.reshape(1, -1), w4c, b4.reshape(1, -1),
      wc1, bc1.reshape(1, -1), wc2, bc2.reshape(1, -1))


def kernel(x1, edge_index1, edge_attr1, x2, edge_index2, edge_attr2,
           W1, b1, W4, b4, Wc1, bc1, Wc2, bc2):
    g = x1.shape[0]
    ei1 = edge_index1.astype(jnp.int32)
    ei2 = edge_index2.astype(jnp.int32)
    ea1 = edge_attr1.astype(jnp.float32)
    ea2 = edge_attr2.astype(jnp.float32)

    a1 = _sc_build_adjacency(ei1, ea1)
    a2 = _sc_build_adjacency(ei2, ea2)

    w1c = jnp.concatenate([W1[0] - W1[2], W1[1], W1[2]], axis=1)
    w4c = jnp.concatenate([W4[0] - W4[2], W4[1], W4[2]], axis=1)

    out1 = _tc_dense(x1, a1, w1c, b1, w4c, b4, Wc1, bc1, Wc2, bc2)
    out2 = _tc_dense(x2, a2, w1c, b1, w4c, b4, Wc1, bc1, Wc2, bc2)
    return out1, out2


# speedup vs baseline: 1.1988x; 1.0948x over previous
import functools

import jax
import jax.numpy as jnp
from jax import lax
from jax.experimental import pallas as pl
from jax.experimental.pallas import tpu as pltpu
from jax.experimental.pallas import tpu_sc as plsc

N = 268
E = 8576
L = 16
NP = 272


def _sc_build_adjacency(ei, ea):
    g = ea.shape[0]
    info = plsc.get_sparse_core_info()
    nc, ns = info.num_cores, info.num_subcores
    assert nc * ns == g

    mesh = plsc.VectorSubcoreMesh(core_axis_name="c", subcore_axis_name="s")

    @functools.partial(
        pl.kernel,
        mesh=mesh,
        out_type=jax.ShapeDtypeStruct((g * NP, NP), jnp.float32),
        scratch_types=[
            pltpu.VMEM((E,), jnp.int32),
            pltpu.VMEM((E,), jnp.int32),
            pltpu.VMEM((E,), jnp.float32),
            pltpu.VMEM((NP, NP), jnp.float32),
            pltpu.SemaphoreType.DMA,
        ],
        compiler_params=pltpu.CompilerParams(needs_layout_passes=False),
    )
    def scatter_kernel(ei_h, ea_h, o_h, src_v, dst_v, ew_v, a_v, sem):
        w = lax.axis_index("s") * nc + lax.axis_index("c")
        zeros16 = jnp.zeros((L,), jnp.float32)

        cps = [pltpu.async_copy(ei_h.at[w, 0], src_v, sem),
               pltpu.async_copy(ei_h.at[w, 1], dst_v, sem),
               pltpu.async_copy(ea_h.at[w], ew_v, sem)]

        @plsc.parallel_loop(0, NP)
        def _zero(i):
            for j in range(NP // L):
                a_v[i, pl.ds(j * L, L)] = zeros16

        for cp in cps:
            cp.wait()

        @plsc.parallel_loop(0, E // L, unroll=4)
        def _scat(i):
            s = src_v[pl.ds(i * L, L)]
            d = dst_v[pl.ds(i * L, L)]
            v = ew_v[pl.ds(i * L, L)]
            plsc.addupdate_scatter(a_v, [d, s], v)

        pltpu.sync_copy(a_v, o_h.at[pl.ds(w * NP, NP)])

    return scatter_kernel(ei, ea)


def _tc_body(x_ref, a_ref, w1_ref, b1_ref, w4_ref, b4_ref, wc1_ref, bc1_ref,
             wc2_ref, bc2_ref, out_ref):
    f32 = jnp.float32
    hi = lax.Precision.DEFAULT

    def mm(p, q):
        return lax.dot_general(p, q, (((1,), (0,)), ((), ())),
                               precision=hi, preferred_element_type=f32)

    gb = x_ref.shape[0]
    ks = range(gb)
    xs = [x_ref[k] for k in ks]
    avs = [a_ref[k * NP:k * NP + N, :N] for k in ks]

    s_mats = []
    for k in ks:
        deg_r = jnp.sum(avs[k], axis=0, keepdims=True)
        dis_r = jnp.where(deg_r > 0,
                          lax.rsqrt(jnp.where(deg_r > 0, deg_r, 1.0)), 0.0)
        dis_c = jnp.transpose(dis_r)
        s_mats.append((dis_c * avs[k]) * dis_r)

    p = [mm(xs[k], w1_ref[...]) for k in ks]
    sp2 = [mm(s_mats[k], p[k][:, 64:96]) for k in ks]
    sall = [mm(s_mats[k], p[k][:, 32:64] - 2.0 * sp2[k]) for k in ks]
    h = [jnp.maximum(p[k][:, 0:32] - sall[k] + b1_ref[...], 0.0) for k in ks]

    q = [mm(h[k], w4_ref[...]) for k in ks]
    sq2 = [mm(s_mats[k], q[k][:, 64:96]) for k in ks]
    sall2 = [mm(s_mats[k], q[k][:, 32:64] - 2.0 * sq2[k]) for k in ks]
    z = [q[k][:, 0:32] - sall2[k] + b4_ref[...] for k in ks]

    zc = [lax.dot_general(z[k], wc1_ref[...], (((0,), (0,)), ((), ())),
                          precision=hi, preferred_element_type=f32)
          for k in ks]
    zc = [jnp.maximum(zc[k] + bc1_ref[...], 0.0) for k in ks]
    for k in ks:
        out_ref[k] = mm(zc[k], wc2_ref[...]) + bc2_ref[...]


def _tc_dense(x_all, a_all, w1c, b1, w4c, b4, wc1, bc1, wc2, bc2,
              interpret=False):
    g = x_all.shape[0]
    nclass = 32
    nout = wc2.shape[-1]

    gb = 8
    full = lambda shape: pl.BlockSpec(shape, lambda i: (0,) * len(shape))
    grid_spec = pl.GridSpec(
        grid=(g // gb,),
        in_specs=[
            pl.BlockSpec((gb, N, N), lambda i: (i, 0, 0)),
            pl.BlockSpec((gb * NP, NP), lambda i: (i, 0)),
            full(w1c.shape),
            full((1, b1.shape[0])),
            full(w4c.shape),
            full((1, b4.shape[0])),
            full(wc1.shape),
            full((1, bc1.shape[0])),
            full(wc2.shape),
            full((1, bc2.shape[0])),
        ],
        out_specs=pl.BlockSpec((gb, nclass, nout), lambda i: (i, 0, 0)),
    )
    return pl.pallas_call(
        _tc_body,
        grid_spec=grid_spec,
        out_shape=jax.ShapeDtypeStruct((g, nclass, nout), jnp.float32),
        interpret=interpret,
    )(x_all, a_all, w1c, b1.reshape(1, -1), w4c, b4.reshape(1, -1),
      wc1, bc1.reshape(1, -1), wc2, bc2.reshape(1, -1))


def kernel(x1, edge_index1, edge_attr1, x2, edge_index2, edge_attr2,
           W1, b1, W4, b4, Wc1, bc1, Wc2, bc2):
    g = x1.shape[0]
    ei1 = edge_index1.astype(jnp.int32)
    ei2 = edge_index2.astype(jnp.int32)
    ea1 = edge_attr1.astype(jnp.float32)
    ea2 = edge_attr2.astype(jnp.float32)

    a1 = _sc_build_adjacency(ei1, ea1)
    a2 = _sc_build_adjacency(ei2, ea2)

    w1c = jnp.concatenate([W1[0] - W1[2], W1[1], W1[2]], axis=1)
    w4c = jnp.concatenate([W4[0] - W4[2], W4[1], W4[2]], axis=1)

    out1 = _tc_dense(x1, a1, w1c, b1, w4c, b4, Wc1, bc1, Wc2, bc2)
    out2 = _tc_dense(x2, a2, w1c, b1, w4c, b4, Wc1, bc1, Wc2, bc2)
    return out1, out2
